# baseline (speedup 1.0000x reference)
import functools

import jax
import jax.numpy as jnp
from jax import lax
from jax.experimental import pallas as pl
from jax.experimental.pallas import tpu as pltpu
from jax.experimental.pallas import tpu_sc as plsc

_NUM_CORES = 2
_NUM_SUBCORES = 16
_NUM_WORKERS = _NUM_CORES * _NUM_SUBCORES

_CHUNK = 128
_NBUF = 5
_LEAD = 3
_LANES = 16


@functools.partial(jax.jit, static_argnames=("seq", "bt_n", "embed_dim"))
def _sc_gather(idxT2, table, *, seq, bt_n, embed_dim):
    n_blocks = idxT2.shape[0]
    blocks_per_w = n_blocks // _NUM_WORKERS
    dt_n = embed_dim // 8

    mesh = plsc.VectorSubcoreMesh(core_axis_name="c", subcore_axis_name="s")

    @functools.partial(
        pl.kernel,
        out_type=jax.ShapeDtypeStruct((seq, dt_n, bt_n, 8, _CHUNK),
                                      jnp.float32),
        mesh=mesh,
        scratch_types=[
            pltpu.VMEM((blocks_per_w, _CHUNK), jnp.int32),
            pltpu.VMEM((_NBUF, _CHUNK, embed_dim), jnp.float32),
            pltpu.VMEM((_NBUF, dt_n, 8, _CHUNK), jnp.float32),
            pltpu.SemaphoreType.DMA((_NBUF,)),
            pltpu.SemaphoreType.DMA((_NBUF,)),
        ],
        compiler_params=pltpu.CompilerParams(
            use_tc_tiling_on_sc=False, needs_layout_passes=False),
    )
    def k(idx_hbm, table_hbm, out_hbm, idx_v, rows_v, t_v, gsem, wsem):
        wid = lax.axis_index("s") * _NUM_CORES + lax.axis_index("c")
        base = wid * blocks_per_w

        pltpu.sync_copy(idx_hbm.at[pl.ds(base, blocks_per_w)], idx_v)

        lane = lax.iota(jnp.int32, _LANES)

        def gather(i, b):
            pltpu.async_copy(
                table_hbm.at[idx_v.at[i]], rows_v.at[b], gsem.at[b])

        def gather_wait(i, b):
            pltpu.make_async_copy(
                table_hbm.at[idx_v.at[i]], rows_v.at[b], gsem.at[b]).wait()

        def wb_dst(i):
            j = base + i
            s = j // bt_n
            bt = lax.rem(j, bt_n)
            return out_hbm.at[s, :, bt]

        def writeback(i, b):
            pltpu.async_copy(t_v.at[b], wb_dst(i), wsem.at[b])

        def writeback_wait(i, b):
            pltpu.make_async_copy(t_v.at[b], wb_dst(i), wsem.at[b]).wait()

        def transpose(b):
            def tbody(dt, carry):
                for di in range(8):
                    d = dt * 8 + di
                    col = jnp.full((_LANES,), d, jnp.int32)
                    for g in range(_CHUNK // _LANES):
                        v = plsc.load_gather(
                            rows_v.at[b], [lane + g * _LANES, col])
                        t_v[b, dt, di, pl.ds(g * _LANES, _LANES)] = v
                return carry

            lax.fori_loop(0, dt_n, tbody, 0)

        for i in range(_LEAD):
            gather(i, i % _NBUF)

        def body(B, carry):
            nxt = B + _LEAD

            @pl.when(nxt < blocks_per_w)
            def _():
                gather(nxt, lax.rem(nxt, _NBUF))

            b = lax.rem(B, _NBUF)
            gather_wait(B, b)

            @pl.when(B >= _NBUF)
            def _():
                writeback_wait(B - _NBUF, b)

            transpose(b)
            writeback(B, b)
            return carry

        lax.fori_loop(0, blocks_per_w, body, 0, unroll=_NBUF)

        for i in range(blocks_per_w - _NBUF, blocks_per_w):
            writeback_wait(i, i % _NBUF)

    return k(idxT2, table)


def kernel(inputs, table):
    batch, seq = inputs.shape
    vocab, embed_dim = table.shape
    bt_n = batch // _CHUNK

    idxT2 = inputs.T.reshape(seq * bt_n, _CHUNK).astype(jnp.int32)

    tflat = lax.optimization_barrier(table.reshape(-1))
    t2 = tflat.reshape(vocab, embed_dim)

    out5 = _sc_gather(idxT2, t2, seq=seq, bt_n=bt_n, embed_dim=embed_dim)
    return out5.transpose(2, 4, 0, 1, 3).reshape(batch, seq, embed_dim)


# speedup vs baseline: 1.0058x; 1.0058x over previous
import functools

import jax
import jax.numpy as jnp
from jax import lax
from jax.experimental import pallas as pl
from jax.experimental.pallas import tpu as pltpu
from jax.experimental.pallas import tpu_sc as plsc

_NUM_CORES = 2
_NUM_SUBCORES = 16
_NUM_WORKERS = _NUM_CORES * _NUM_SUBCORES

_CHUNK = 128
_NBUF = 5
_LEAD = 3
_LANES = 16


@functools.partial(jax.jit, static_argnames=("seq", "bt_n", "embed_dim"))
def _sc_gather(idxT2, table, *, seq, bt_n, embed_dim):
    n_blocks = idxT2.shape[0]
    blocks_per_w = n_blocks // _NUM_WORKERS
    dt_n = embed_dim // 8

    mesh = plsc.VectorSubcoreMesh(core_axis_name="c", subcore_axis_name="s")

    @functools.partial(
        pl.kernel,
        out_type=jax.ShapeDtypeStruct((seq, dt_n, bt_n, 8, _CHUNK),
                                      jnp.float32),
        mesh=mesh,
        scratch_types=[
            pltpu.VMEM((blocks_per_w, _CHUNK), jnp.int32),
            pltpu.VMEM((_NBUF, _CHUNK, embed_dim), jnp.float32),
            pltpu.VMEM((_NBUF, dt_n, 8, _CHUNK), jnp.float32),
            pltpu.SemaphoreType.DMA((_NBUF,)),
            pltpu.SemaphoreType.DMA((_NBUF,)),
        ],
        compiler_params=pltpu.CompilerParams(
            use_tc_tiling_on_sc=False, needs_layout_passes=False),
    )
    def k(idx_hbm, table_hbm, out_hbm, idx_v, rows_v, t_v, gsem, wsem):
        wid = lax.axis_index("s") * _NUM_CORES + lax.axis_index("c")
        base = wid * blocks_per_w

        pltpu.sync_copy(idx_hbm.at[pl.ds(base, blocks_per_w)], idx_v)

        lane = lax.iota(jnp.int32, _LANES)

        def gather(i, b):
            pltpu.async_copy(
                table_hbm.at[idx_v.at[i]], rows_v.at[b], gsem.at[b])

        def gather_wait(i, b):
            pltpu.make_async_copy(
                table_hbm.at[idx_v.at[i]], rows_v.at[b], gsem.at[b]).wait()

        def wb_dst(i):
            j = base + i
            s = j // bt_n
            bt = lax.rem(j, bt_n)
            return out_hbm.at[s, :, bt]

        def writeback(i, b):
            pltpu.async_copy(t_v.at[b], wb_dst(i), wsem.at[b])

        def writeback_wait(i, b):
            pltpu.make_async_copy(t_v.at[b], wb_dst(i), wsem.at[b]).wait()

        rowv = [lane + g * _LANES for g in range(_CHUNK // _LANES)]

        def transpose(b):
            for dt in range(dt_n):
                for di in range(8):
                    col = jnp.full((_LANES,), dt * 8 + di, jnp.int32)
                    for g in range(_CHUNK // _LANES):
                        v = plsc.load_gather(rows_v.at[b], [rowv[g], col])
                        t_v[b, dt, di, pl.ds(g * _LANES, _LANES)] = v

        for i in range(_LEAD):
            gather(i, i % _NBUF)

        def body(B, carry):
            nxt = B + _LEAD

            @pl.when(nxt < blocks_per_w)
            def _():
                gather(nxt, lax.rem(nxt, _NBUF))

            b = lax.rem(B, _NBUF)
            gather_wait(B, b)

            @pl.when(B >= _NBUF)
            def _():
                writeback_wait(B - _NBUF, b)

            transpose(b)
            writeback(B, b)
            return carry

        lax.fori_loop(0, blocks_per_w, body, 0)

        for i in range(blocks_per_w - _NBUF, blocks_per_w):
            writeback_wait(i, i % _NBUF)

    return k(idxT2, table)


def kernel(inputs, table):
    batch, seq = inputs.shape
    vocab, embed_dim = table.shape
    bt_n = batch // _CHUNK

    idxT2 = inputs.T.reshape(seq * bt_n, _CHUNK).astype(jnp.int32)

    tflat = lax.optimization_barrier(table.reshape(-1))
    t2 = tflat.reshape(vocab, embed_dim)

    out5 = _sc_gather(idxT2, t2, seq=seq, bt_n=bt_n, embed_dim=embed_dim)
    return out5.transpose(2, 4, 0, 1, 3).reshape(batch, seq, embed_dim)


# speedup vs baseline: 1.1618x; 1.1551x over previous
import functools

import jax
import jax.numpy as jnp
from jax import lax
from jax.experimental import pallas as pl
from jax.experimental.pallas import tpu as pltpu
from jax.experimental.pallas import tpu_sc as plsc

_NUM_CORES = 2
_NUM_SUBCORES = 16
_NUM_WORKERS = _NUM_CORES * _NUM_SUBCORES

_CHUNK = 128
_NBUF = 5
_LEAD = 3
_LANES = 16


@functools.partial(jax.jit, static_argnames=("seq", "bt_n", "embed_dim"))
def _sc_gather(idxT2, table, *, seq, bt_n, embed_dim):
    n_blocks = idxT2.shape[0]
    blocks_per_w = n_blocks // _NUM_WORKERS
    dt_n = embed_dim // 8

    mesh = plsc.VectorSubcoreMesh(core_axis_name="c", subcore_axis_name="s")

    @functools.partial(
        pl.kernel,
        out_type=jax.ShapeDtypeStruct((seq, dt_n, bt_n, 8, _CHUNK),
                                      jnp.float32),
        mesh=mesh,
        scratch_types=[
            pltpu.VMEM((blocks_per_w, _CHUNK), jnp.int32),
            pltpu.VMEM((_NBUF, _CHUNK, embed_dim), jnp.float32),
            pltpu.VMEM((_NBUF, dt_n, 8, _CHUNK), jnp.float32),
            pltpu.SemaphoreType.DMA((_NBUF,)),
            pltpu.SemaphoreType.DMA((_NBUF,)),
        ],
        compiler_params=pltpu.CompilerParams(
            use_tc_tiling_on_sc=False, needs_layout_passes=False),
    )
    def k(idx_hbm, table_hbm, out_hbm, idx_v, rows_v, t_v, gsem, wsem):
        wid = lax.axis_index("s") * _NUM_CORES + lax.axis_index("c")
        base = wid * blocks_per_w

        pltpu.sync_copy(idx_hbm.at[pl.ds(base, blocks_per_w)], idx_v)

        lane = lax.iota(jnp.int32, _LANES)

        def gather(i, b):
            pltpu.async_copy(
                table_hbm.at[idx_v.at[i]], rows_v.at[b], gsem.at[b])

        def gather_wait(i, b):
            pltpu.make_async_copy(
                table_hbm.at[idx_v.at[i]], rows_v.at[b], gsem.at[b]).wait()

        def wb_dst(i):
            j = base + i
            s = j // bt_n
            bt = lax.rem(j, bt_n)
            return out_hbm.at[s, :, bt]

        def writeback(i, b):
            pltpu.async_copy(t_v.at[b], wb_dst(i), wsem.at[b])

        def writeback_wait(i, b):
            pltpu.make_async_copy(t_v.at[b], wb_dst(i), wsem.at[b]).wait()

        rowv = [lane + g * _LANES for g in range(_CHUNK // _LANES)]

        def transpose(b):
            ng = _CHUNK // _LANES
            for dt in range(dt_n):
                for di in range(8):
                    col = jnp.full((_LANES,), dt * 8 + di, jnp.int32)
                    vs = [plsc.load_gather(rows_v.at[b], [rowv[g], col])
                          for g in range(ng)]
                    for g in range(ng):
                        t_v[b, dt, di, pl.ds(g * _LANES, _LANES)] = vs[g]

        for i in range(_LEAD):
            gather(i, i % _NBUF)

        def body(B, carry):
            nxt = B + _LEAD

            @pl.when(nxt < blocks_per_w)
            def _():
                gather(nxt, lax.rem(nxt, _NBUF))

            b = lax.rem(B, _NBUF)
            gather_wait(B, b)

            @pl.when(B >= _NBUF)
            def _():
                writeback_wait(B - _NBUF, b)

            transpose(b)
            writeback(B, b)
            return carry

        lax.fori_loop(0, blocks_per_w, body, 0)

        for i in range(blocks_per_w - _NBUF, blocks_per_w):
            writeback_wait(i, i % _NBUF)

    return k(idxT2, table)


def kernel(inputs, table):
    batch, seq = inputs.shape
    vocab, embed_dim = table.shape
    bt_n = batch // _CHUNK

    idxT2 = inputs.T.reshape(seq * bt_n, _CHUNK).astype(jnp.int32)

    tflat = lax.optimization_barrier(table.reshape(-1))
    t2 = tflat.reshape(vocab, embed_dim)

    out5 = _sc_gather(idxT2, t2, seq=seq, bt_n=bt_n, embed_dim=embed_dim)
    return out5.transpose(2, 4, 0, 1, 3).reshape(batch, seq, embed_dim)


# speedup vs baseline: 1.5306x; 1.3174x over previous
import functools

import jax
import jax.numpy as jnp
from jax import lax
from jax.experimental import pallas as pl
from jax.experimental.pallas import tpu as pltpu
from jax.experimental.pallas import tpu_sc as plsc

_NUM_CORES = 2
_NUM_SUBCORES = 16
_NUM_WORKERS = _NUM_CORES * _NUM_SUBCORES

_CHUNK = 128
_NBUF = 5
_LEAD = 3


@functools.partial(jax.jit, static_argnames=("seq", "bt_n", "embed_dim"))
def _sc_gather(idxT2, table, *, seq, bt_n, embed_dim):
    n_blocks = idxT2.shape[0]
    blocks_per_w = n_blocks // _NUM_WORKERS

    mesh = plsc.VectorSubcoreMesh(core_axis_name="c", subcore_axis_name="s")

    @functools.partial(
        pl.kernel,
        out_type=jax.ShapeDtypeStruct((seq, bt_n, _CHUNK, embed_dim),
                                      jnp.float32),
        mesh=mesh,
        scratch_types=[
            pltpu.VMEM((blocks_per_w, _CHUNK), jnp.int32),
            pltpu.VMEM((_NBUF, _CHUNK, embed_dim), jnp.float32),
            pltpu.SemaphoreType.DMA((_NBUF,)),
            pltpu.SemaphoreType.DMA((_NBUF,)),
        ],
        compiler_params=pltpu.CompilerParams(
            use_tc_tiling_on_sc=False, needs_layout_passes=False),
    )
    def k(idx_hbm, table_hbm, out_hbm, idx_v, rows_v, gsem, wsem):
        wid = lax.axis_index("s") * _NUM_CORES + lax.axis_index("c")
        base = wid * blocks_per_w

        pltpu.sync_copy(idx_hbm.at[pl.ds(base, blocks_per_w)], idx_v)

        def gather(i, b):
            pltpu.async_copy(
                table_hbm.at[idx_v.at[i]], rows_v.at[b], gsem.at[b])

        def gather_wait(i, b):
            pltpu.make_async_copy(
                table_hbm.at[idx_v.at[i]], rows_v.at[b], gsem.at[b]).wait()

        def wb_dst(i):
            j = base + i
            return out_hbm.at[j // bt_n, lax.rem(j, bt_n)]

        def writeback(i, b):
            pltpu.async_copy(rows_v.at[b], wb_dst(i), wsem.at[b])

        def writeback_wait(i, b):
            pltpu.make_async_copy(rows_v.at[b], wb_dst(i), wsem.at[b]).wait()

        for i in range(_LEAD):
            gather(i, i % _NBUF)

        def body(B, carry):
            nxt = B + _LEAD

            @pl.when(nxt < blocks_per_w)
            def _():
                gather(nxt, lax.rem(nxt, _NBUF))

            b = lax.rem(B, _NBUF)
            gather_wait(B, b)

            @pl.when(B >= _NBUF)
            def _():
                writeback_wait(B - _NBUF, b)

            writeback(B, b)
            return carry

        lax.fori_loop(0, blocks_per_w, body, 0)

        for i in range(blocks_per_w - _NBUF, blocks_per_w):
            writeback_wait(i, i % _NBUF)

    return k(idxT2, table)


def kernel(inputs, table):
    batch, seq = inputs.shape
    vocab, embed_dim = table.shape
    bt_n = batch // _CHUNK

    idxT2 = inputs.T.reshape(seq * bt_n, _CHUNK).astype(jnp.int32)

    tflat = lax.optimization_barrier(table.reshape(-1))
    t2 = tflat.reshape(vocab, embed_dim)

    out4 = _sc_gather(idxT2, t2, seq=seq, bt_n=bt_n, embed_dim=embed_dim)
    return out4.transpose(1, 2, 0, 3).reshape(batch, seq, embed_dim)
